# Initial kernel scaffold; baseline (speedup 1.0000x reference)
import jax
import jax.numpy as jnp
from jax.experimental import pallas as pl


def kernel(x, edge_index, idx, edge_type, edge_weight, proj_W, proj_b, bn_gamma, bn_beta, W_rel_0, Win_W_0, Win_b_0, Wout_W_0, Wout_b_0, W_rel_1, Win_W_1, Win_b_1, Wout_W_1, Wout_b_1):
    raise NotImplementedError("write your pallas kernel here")


# speedup vs baseline: 1.0019x; 1.0019x over previous
import jax
import jax.numpy as jnp
from jax.experimental import pallas as pl

N = 10000
E = 320000
D = 128
REL = 4


def _layer(cont, row, col, edge_type, dis, Win_W, Win_b, W_rel, Wout_W, Wout_b):
    xl = cont @ Win_W + Win_b
    z = jnp.einsum('nd,rde->rne', xl, W_rel)
    u = dis[:, None] * xl
    res = z[edge_type, col]
    xj = u[col]
    msg_gcn = dis[:, None] * jax.ops.segment_sum(xj, row, num_segments=N)
    ex = jnp.exp(res)
    num = jax.ops.segment_sum(res * ex, row, num_segments=N)
    den = jax.ops.segment_sum(ex, row, num_segments=N)
    msg = num / (den + 1e-16)
    return (msg_gcn + 0.5 * jax.nn.relu(msg)) @ Wout_W + Wout_b


def kernel(x, edge_index, idx, edge_type, edge_weight, proj_W, proj_b, bn_gamma, bn_beta, W_rel_0, Win_W_0, Win_b_0, Wout_W_0, Wout_b_0, W_rel_1, Win_W_1, Win_b_1, Wout_W_1, Wout_b_1):
    h = x @ proj_W + proj_b
    mean = jnp.mean(h, axis=0)
    var = jnp.var(h, axis=0)
    h = (h - mean) * jax.lax.rsqrt(var + 1e-5) * bn_gamma + bn_beta
    cont = jax.nn.relu(h)

    row = edge_index[0]
    col = edge_index[1]
    deg = jax.ops.segment_sum(jnp.ones((E,), jnp.float32), col, num_segments=N)
    dis = jnp.where(deg > 0, jax.lax.rsqrt(jnp.maximum(deg, 1e-12)), 0.0)

    cont = _layer(cont, row, col, edge_type, dis, Win_W_0, Win_b_0, W_rel_0, Wout_W_0, Wout_b_0)
    cont = _layer(cont, row, col, edge_type, dis, Win_W_1, Win_b_1, W_rel_1, Wout_W_1, Wout_b_1)
    cont = jax.nn.gelu(cont, approximate=False)
    return cont[idx]


# speedup vs baseline: 7.3587x; 7.3445x over previous
import functools

import jax
import jax.numpy as jnp
from jax import lax
from jax.experimental import pallas as pl
from jax.experimental.pallas import tpu as pltpu
from jax.experimental.pallas import tpu_sc as plsc

N = 10000
E = 320000
D = 128
REL = 4
NIDX = 2048

NC = 2
NS = 16

CH = 128
EPT = 20000
NB = 4
BKT = N // NB
ACC_R = 2560
ZPT_A = ACC_R // NS
SCH = 2000
NSCH = EPT // SCH
CAP = 20224
ACC_ROWS = 10112
ZPT = ACC_ROWS // NS

DEG_CH = 80
DEG_PER_W = E // (NC * NS)
DEG_NCHUNK = DEG_PER_W // DEG_CH

BLK = 400
GRID = N // BLK


def _dis_from_deg(deg_blk):
    deg = jnp.sum(deg_blk, axis=1, keepdims=True)
    return jnp.where(deg > 0, lax.rsqrt(jnp.maximum(deg, 1e-12)), 0.0)


def _proj_body(x_ref, pw_ref, pb_ref, h_ref, st_ref, acc_ref):
    i = pl.program_id(0)
    h = lax.dot_general(x_ref[...], pw_ref[...], (((1,), (0,)), ((), ())),
                        preferred_element_type=jnp.float32) + pb_ref[...]
    h_ref[...] = h

    @pl.when(i == 0)
    def _():
        acc_ref[...] = jnp.zeros_like(acc_ref)

    acc_ref[0:1, :] += jnp.sum(h, axis=0, keepdims=True)
    acc_ref[1:2, :] += jnp.sum(h * h, axis=0, keepdims=True)

    @pl.when(i == GRID - 1)
    def _():
        st_ref[...] = acc_ref[...]


def _mm(a, b):
    return lax.dot_general(a, b, (((1,), (0,)), ((), ())),
                           preferred_element_type=jnp.float32)


def _tables_out(x1, dis, wrel_ref, tbl_ref):
    u = dis * x1
    for t in range(REL):
        z = _mm(x1, wrel_ref[t])
        tbl_ref[0, t] = jnp.concatenate([u[:, :64], z[:, :64]], axis=1)
        tbl_ref[1, t] = jnp.concatenate([u[:, 64:], z[:, 64:]], axis=1)


def _prologue_tables_body(h_ref, st_ref, g_ref, b_ref, win_ref, winb_ref,
                          wrel_ref, deg_ref, tbl_ref):
    mean = st_ref[0:1, :] / N
    var = st_ref[1:2, :] / N - mean * mean
    hn = (h_ref[...] - mean) * lax.rsqrt(var + 1e-5) * g_ref[...] + b_ref[...]
    cont = jnp.maximum(hn, 0.0)
    x1 = _mm(cont, win_ref[...]) + winb_ref[...]
    _tables_out(x1, _dis_from_deg(deg_ref[...]), wrel_ref, tbl_ref)


def _epilogue(aA_ref, aB_ref, deg_ref, wout_ref, woutb_ref):
    mg = jnp.concatenate([aA_ref[0, :, :64], aA_ref[1, :, :64]], axis=1)
    den = jnp.concatenate([aA_ref[0, :, 64:], aA_ref[1, :, 64:]], axis=1)
    num = jnp.concatenate([aB_ref[0, :, :64], aB_ref[1, :, :64]], axis=1)
    msg_gcn = _dis_from_deg(deg_ref[...]) * mg
    msg = num / (den + 1e-16)
    y = msg_gcn + 0.5 * jnp.maximum(msg, 0.0)
    return _mm(y, wout_ref[...]) + woutb_ref[...]


def _mid_body(aA_ref, aB_ref, deg_ref, wout_ref, woutb_ref,
              win_ref, winb_ref, wrel_ref, tbl_ref):
    cont = _epilogue(aA_ref, aB_ref, deg_ref, wout_ref, woutb_ref)
    x2 = _mm(cont, win_ref[...]) + winb_ref[...]
    _tables_out(x2, _dis_from_deg(deg_ref[...]), wrel_ref, tbl_ref)


def _erf(x):
    s = jnp.sign(x)
    a = jnp.abs(x)
    t = 1.0 / (1.0 + 0.3275911 * a)
    poly = t * (0.254829592 + t * (-0.284496736 + t * (1.421413741
               + t * (-1.453152027 + t * 1.061405429))))
    return s * (1.0 - poly * jnp.exp(-a * a))


def _final_body(aA_ref, aB_ref, deg_ref, wout_ref, woutb_ref, g_ref):
    cont = _epilogue(aA_ref, aB_ref, deg_ref, wout_ref, woutb_ref)
    g_ref[...] = cont * 0.5 * (1.0 + _erf(cont * 0.7071067811865476))


def _row_spec(shape_prefix=(), minor=D):
    nd = len(shape_prefix)
    return pl.BlockSpec(shape_prefix + (BLK, minor),
                        lambda i: (0,) * nd + (i, 0))


def _full(shape):
    return pl.BlockSpec(shape, lambda i: (0,) * len(shape))


def _tc_calls():
    f32 = jnp.float32
    proj = pl.pallas_call(
        _proj_body,
        grid=(GRID,),
        in_specs=[_row_spec(), _full((D, D)), _full((1, D))],
        out_specs=[_row_spec(), _full((8, D))],
        out_shape=[jax.ShapeDtypeStruct((N, D), f32),
                   jax.ShapeDtypeStruct((8, D), f32)],
        scratch_shapes=[pltpu.VMEM((8, D), f32)],
    )
    tbl_spec = pl.BlockSpec((2, REL, BLK, D), lambda i: (0, 0, i, 0))
    tbl_shape = jax.ShapeDtypeStruct((2, REL, N, D), f32)
    prologue_tables = pl.pallas_call(
        _prologue_tables_body,
        grid=(GRID,),
        in_specs=[_row_spec(), _full((8, D)), _full((1, D)), _full((1, D)),
                  _full((D, D)), _full((1, D)), _full((REL, D, D)),
                  _row_spec((), 32)],
        out_specs=tbl_spec,
        out_shape=tbl_shape,
    )
    acc_specs = [_row_spec((2,), D), _row_spec((2,), D), _row_spec((), 32)]
    mid = pl.pallas_call(
        _mid_body,
        grid=(GRID,),
        in_specs=acc_specs + [_full((D, D)), _full((1, D)), _full((D, D)),
                              _full((1, D)), _full((REL, D, D))],
        out_specs=tbl_spec,
        out_shape=tbl_shape,
    )
    final = pl.pallas_call(
        _final_body,
        grid=(GRID,),
        in_specs=acc_specs + [_full((D, D)), _full((1, D))],
        out_specs=_row_spec(),
        out_shape=jax.ShapeDtypeStruct((N, D), f32),
    )
    return proj, prologue_tables, mid, final


def _fill(ref, rows, value):
    width = ref.shape[1]
    val = jnp.full((16,), value, dtype=ref.dtype)

    def body(r, _):
        for j in range(width // 16):
            ref[r, pl.ds(j * 16, 16)] = val
        return 0

    lax.fori_loop(0, rows, body, 0)


def _deg_body(col_hbm, deg_out, cstage, hist):
    c = lax.axis_index("c")
    s = lax.axis_index("s")
    w = c * NS + s

    def zero(r, _):
        hist[pl.ds(r * 16, 16)] = jnp.zeros((16,), jnp.float32)
        return 0

    lax.fori_loop(0, N // 16, zero, 0)
    one = jnp.ones((16,), jnp.float32)
    lane = lax.iota(jnp.int32, 16)
    base0 = w * DEG_PER_W

    def body(k, _):
        pltpu.sync_copy(col_hbm.at[pl.ds(base0 + k * SCH, SCH)], cstage)

        def grp(j, _):
            o = pl.multiple_of(j * 16, 16)
            c16 = cstage[pl.ds(o, 16)]
            for i in range(16):
                plsc.addupdate_scatter(hist, [c16], one, mask=lane == i)
            return 0

        lax.fori_loop(0, SCH // 16, grp, 0)
        return 0

    lax.fori_loop(0, DEG_PER_W // SCH, body, 0)
    pltpu.sync_copy(hist, deg_out.at[pl.ds(w * N, N)])


def _edge_body(tbl_hbm, row_hbm, col_hbm, typ_hbm, z128_hbm,
               outA, outB,
               rstage, cstage, tstage,
               lrow_b, gidx_b,
               row_v, gidx_v, buf, rexbuf,
               accA, accB, sem1):
    c = lax.axis_index("c")
    s = lax.axis_index("s")
    goff = c * (REL * N)
    ebase = s * EPT
    _fill(rexbuf, CH, 0.0)

    for b in range(NB):
        lo = b * BKT

        def comp_chunk(k, p, lo=lo):
            base = ebase + k * SCH
            pltpu.sync_copy(row_hbm.at[pl.ds(base, SCH)], rstage)
            pltpu.sync_copy(col_hbm.at[pl.ds(base, SCH)], cstage)
            pltpu.sync_copy(typ_hbm.at[pl.ds(base, SCH)], tstage)

            def grp(j, p):
                o = pl.multiple_of(j * 16, 16)
                r16 = rstage[pl.ds(o, 16)] - lo
                c16 = cstage[pl.ds(o, 16)]
                t16 = tstage[pl.ds(o, 16)]
                g16 = t16 * N + c16 + goff
                m = jnp.logical_and(r16 >= 0, r16 < BKT)
                mi = jnp.where(m, 1, 0)
                incl = plsc.cumsum(mi)
                pos = p + (incl - mi)
                plsc.store_scatter(lrow_b, [pos], r16, mask=m)
                plsc.store_scatter(gidx_b, [pos], g16, mask=m)
                return p + jnp.max(incl)

            return lax.fori_loop(0, SCH // 16, grp, p)

        cnt = lax.fori_loop(0, NSCH, comp_chunk, 0)

        trash = jnp.full((16,), BKT, jnp.int32)
        gpad = jnp.full((16,), 0, jnp.int32) + goff
        for j in range(CH // 16):
            lrow_b[pl.ds(cnt + j * 16, 16)] = trash
            gidx_b[pl.ds(cnt + j * 16, 16)] = gpad

        pltpu.sync_copy(z128_hbm, accA.at[pl.ds(s * ZPT_A, ZPT_A)])
        pltpu.sync_copy(z128_hbm, accB.at[pl.ds(s * ZPT_A, ZPT_A)])
        plsc.subcore_barrier()

        def chunk(k, _):
            o = k * CH

            def cp(j, _):
                oj = pl.multiple_of(j * 16, 16)
                row_v[pl.ds(oj, 16)] = lrow_b[pl.ds(o + oj, 16)]
                gidx_v[pl.ds(oj, 16)] = gidx_b[pl.ds(o + oj, 16)]
                return 0

            lax.fori_loop(0, CH // 16, cp, 0)
            pltpu.async_copy(tbl_hbm.at[gidx_v], buf, sem1).wait()

            def exp_body(e, _):
                for j in range(4):
                    oj = pl.multiple_of(j * 16, 16)
                    z = buf[e, pl.ds(64 + oj, 16)]
                    ex = jnp.exp(z)
                    buf[e, pl.ds(64 + oj, 16)] = ex
                    rexbuf[e, pl.ds(oj, 16)] = z * ex
                return 0

            lax.fori_loop(0, CH, exp_body, 0)
            pltpu.sync_copy(buf, accA.at[row_v], add=True)
            pltpu.sync_copy(rexbuf, accB.at[row_v], add=True)
            return 0

        nch = (cnt + CH - 1) // CH
        lax.fori_loop(0, nch, chunk, 0)
        plsc.subcore_barrier()
        obase = (c * NB + b) * ACC_R + s * ZPT_A
        pltpu.sync_copy(accA.at[pl.ds(s * ZPT_A, ZPT_A)],
                        outA.at[pl.ds(obase, ZPT_A)])
        pltpu.sync_copy(accB.at[pl.ds(s * ZPT_A, ZPT_A)],
                        outB.at[pl.ds(obase, ZPT_A)])
        plsc.subcore_barrier()


def _gather_body(g_hbm, idx_hbm, out_hbm, idx_v, buf, sem):
    c = lax.axis_index("c")
    s = lax.axis_index("s")
    base = (c * NS + s) * (NIDX // (NC * NS))
    pltpu.sync_copy(idx_hbm.at[pl.ds(base, NIDX // (NC * NS))], idx_v)
    pltpu.async_copy(g_hbm.at[idx_v], buf, sem).wait()
    pltpu.sync_copy(buf, out_hbm.at[pl.ds(base, NIDX // (NC * NS))])


def _sc_calls():
    f32 = jnp.float32
    i32 = jnp.int32
    mesh = plsc.VectorSubcoreMesh(core_axis_name="c", subcore_axis_name="s",
                                  num_cores=NC, num_subcores=NS)
    deg = pl.kernel(
        _deg_body,
        out_type=jax.ShapeDtypeStruct((NC * NS * N,), f32),
        mesh=mesh,
        compiler_params=pltpu.CompilerParams(needs_layout_passes=False),
        scratch_types=[
            pltpu.VMEM((SCH,), i32),
            pltpu.VMEM((N,), f32),
        ],
    )
    edge = pl.kernel(
        _edge_body,
        out_type=(jax.ShapeDtypeStruct((NC * NB * ACC_R, D), f32),
                  jax.ShapeDtypeStruct((NC * NB * ACC_R, D), f32)),
        mesh=mesh,
        compiler_params=pltpu.CompilerParams(needs_layout_passes=False),
        scratch_types=[
            pltpu.VMEM((SCH,), i32), pltpu.VMEM((SCH,), i32),
            pltpu.VMEM((SCH,), i32),
            pltpu.VMEM((CAP,), i32), pltpu.VMEM((CAP,), i32),
            pltpu.VMEM((CH,), i32), pltpu.VMEM((CH,), i32),
            pltpu.VMEM((CH, D), f32), pltpu.VMEM((CH, D), f32),
            pltpu.VMEM_SHARED((ACC_R, D), f32),
            pltpu.VMEM_SHARED((ACC_R, D), f32),
            pltpu.SemaphoreType.DMA,
        ],
    )
    gather = pl.kernel(
        _gather_body,
        out_type=jax.ShapeDtypeStruct((NIDX, D), f32),
        mesh=mesh,
        scratch_types=[
            pltpu.VMEM((NIDX // (NC * NS),), i32),
            pltpu.VMEM((NIDX // (NC * NS), D), f32),
            pltpu.SemaphoreType.DMA,
        ],
    )
    return deg, edge, gather


def kernel(x, edge_index, idx, edge_type, edge_weight, proj_W, proj_b,
           bn_gamma, bn_beta, W_rel_0, Win_W_0, Win_b_0, Wout_W_0, Wout_b_0,
           W_rel_1, Win_W_1, Win_b_1, Wout_W_1, Wout_b_1):
    f32 = jnp.float32
    i32 = jnp.int32
    proj, prologue_tables, mid, final = _tc_calls()
    deg_k, edge_k, gather_k = _sc_calls()

    row = edge_index[0]
    col = edge_index[1]

    deg_out = deg_k(col)
    degp = deg_out.reshape(NC * NS, N).T

    h, stats = proj(x, proj_W, proj_b.reshape(1, D))
    z128 = jnp.zeros((ZPT_A, D), f32)

    def layer(tbl):
        aA, aB = edge_k(tbl.reshape(NC * REL * N, D), row, col, edge_type,
                        z128)

        def halves(a):
            def core(ci):
                o = ci * NB * ACC_R
                return jnp.concatenate(
                    [a[o + b * ACC_R:o + b * ACC_R + BKT] for b in range(NB)])

            return jnp.stack([core(0), core(1)])

        return halves(aA), halves(aB)

    tbl1 = prologue_tables(h, stats, bn_gamma.reshape(1, D),
                           bn_beta.reshape(1, D), Win_W_0,
                           Win_b_0.reshape(1, D), W_rel_0, degp)
    aA, aB = layer(tbl1)
    tbl2 = mid(aA, aB, degp, Wout_W_0, Wout_b_0.reshape(1, D),
               Win_W_1, Win_b_1.reshape(1, D), W_rel_1)
    aA, aB = layer(tbl2)
    g = final(aA, aB, degp, Wout_W_1, Wout_b_1.reshape(1, D))
    return gather_k(g, idx)


# speedup vs baseline: 8.7493x; 1.1890x over previous
import functools

import jax
import jax.numpy as jnp
from jax import lax
from jax.experimental import pallas as pl
from jax.experimental.pallas import tpu as pltpu
from jax.experimental.pallas import tpu_sc as plsc

N = 10000
E = 320000
D = 128
REL = 4
NIDX = 2048

NC = 2
NS = 16

CH = 128
EPT = 20000
NB = 5
BKT = N // NB
ACC_R = 2048
ZPT_A = ACC_R // NS
SCH = 2000
NSCH = EPT // SCH
CAP = 20224
ACC_ROWS = 10112
ZPT = ACC_ROWS // NS

DEG_CH = 80
DEG_PER_W = E // (NC * NS)
DEG_NCHUNK = DEG_PER_W // DEG_CH

BLK = 400
GRID = N // BLK


def _dis_from_deg(deg_blk):
    deg = jnp.sum(deg_blk, axis=1, keepdims=True)
    return jnp.where(deg > 0, lax.rsqrt(jnp.maximum(deg, 1e-12)), 0.0)


def _proj_body(x_ref, pw_ref, pb_ref, h_ref, st_ref, acc_ref):
    i = pl.program_id(0)
    h = lax.dot_general(x_ref[...], pw_ref[...], (((1,), (0,)), ((), ())),
                        preferred_element_type=jnp.float32) + pb_ref[...]
    h_ref[...] = h

    @pl.when(i == 0)
    def _():
        acc_ref[...] = jnp.zeros_like(acc_ref)

    acc_ref[0:1, :] += jnp.sum(h, axis=0, keepdims=True)
    acc_ref[1:2, :] += jnp.sum(h * h, axis=0, keepdims=True)

    @pl.when(i == GRID - 1)
    def _():
        st_ref[...] = acc_ref[...]


def _mm(a, b):
    return lax.dot_general(a, b, (((1,), (0,)), ((), ())),
                           preferred_element_type=jnp.float32)


def _tables_out(x1, dis, wrel_ref, tbl_ref):
    u = dis * x1
    for t in range(REL):
        z = _mm(x1, wrel_ref[t])
        tbl_ref[0, t] = jnp.concatenate([u[:, :64], z[:, :64]], axis=1)
        tbl_ref[1, t] = jnp.concatenate([u[:, 64:], z[:, 64:]], axis=1)


def _prologue_tables_body(h_ref, st_ref, g_ref, b_ref, win_ref, winb_ref,
                          wrel_ref, deg_ref, tbl_ref):
    mean = st_ref[0:1, :] / N
    var = st_ref[1:2, :] / N - mean * mean
    hn = (h_ref[...] - mean) * lax.rsqrt(var + 1e-5) * g_ref[...] + b_ref[...]
    cont = jnp.maximum(hn, 0.0)
    x1 = _mm(cont, win_ref[...]) + winb_ref[...]
    _tables_out(x1, _dis_from_deg(deg_ref[...]), wrel_ref, tbl_ref)


def _epilogue(aA_ref, aB_ref, deg_ref, wout_ref, woutb_ref):
    mg = jnp.concatenate([aA_ref[0, :, :64], aA_ref[1, :, :64]], axis=1)
    den = jnp.concatenate([aA_ref[0, :, 64:], aA_ref[1, :, 64:]], axis=1)
    num = jnp.concatenate([aB_ref[0, :, :64], aB_ref[1, :, :64]], axis=1)
    msg_gcn = _dis_from_deg(deg_ref[...]) * mg
    msg = num / (den + 1e-16)
    y = msg_gcn + 0.5 * jnp.maximum(msg, 0.0)
    return _mm(y, wout_ref[...]) + woutb_ref[...]


def _mid_body(aA_ref, aB_ref, deg_ref, wout_ref, woutb_ref,
              win_ref, winb_ref, wrel_ref, tbl_ref):
    cont = _epilogue(aA_ref, aB_ref, deg_ref, wout_ref, woutb_ref)
    x2 = _mm(cont, win_ref[...]) + winb_ref[...]
    _tables_out(x2, _dis_from_deg(deg_ref[...]), wrel_ref, tbl_ref)


def _erf(x):
    s = jnp.sign(x)
    a = jnp.abs(x)
    t = 1.0 / (1.0 + 0.3275911 * a)
    poly = t * (0.254829592 + t * (-0.284496736 + t * (1.421413741
               + t * (-1.453152027 + t * 1.061405429))))
    return s * (1.0 - poly * jnp.exp(-a * a))


def _final_body(aA_ref, aB_ref, deg_ref, wout_ref, woutb_ref, g_ref):
    cont = _epilogue(aA_ref, aB_ref, deg_ref, wout_ref, woutb_ref)
    g_ref[...] = cont * 0.5 * (1.0 + _erf(cont * 0.7071067811865476))


def _row_spec(shape_prefix=(), minor=D):
    nd = len(shape_prefix)
    return pl.BlockSpec(shape_prefix + (BLK, minor),
                        lambda i: (0,) * nd + (i, 0))


def _full(shape):
    return pl.BlockSpec(shape, lambda i: (0,) * len(shape))


def _tc_calls():
    f32 = jnp.float32
    proj = pl.pallas_call(
        _proj_body,
        grid=(GRID,),
        in_specs=[_row_spec(), _full((D, D)), _full((1, D))],
        out_specs=[_row_spec(), _full((8, D))],
        out_shape=[jax.ShapeDtypeStruct((N, D), f32),
                   jax.ShapeDtypeStruct((8, D), f32)],
        scratch_shapes=[pltpu.VMEM((8, D), f32)],
    )
    tbl_spec = pl.BlockSpec((2, REL, BLK, D), lambda i: (0, 0, i, 0))
    tbl_shape = jax.ShapeDtypeStruct((2, REL, N, D), f32)
    prologue_tables = pl.pallas_call(
        _prologue_tables_body,
        grid=(GRID,),
        in_specs=[_row_spec(), _full((8, D)), _full((1, D)), _full((1, D)),
                  _full((D, D)), _full((1, D)), _full((REL, D, D)),
                  _row_spec((), 32)],
        out_specs=tbl_spec,
        out_shape=tbl_shape,
    )
    acc_specs = [_row_spec((2,), D), _row_spec((2,), D), _row_spec((), 32)]
    mid = pl.pallas_call(
        _mid_body,
        grid=(GRID,),
        in_specs=acc_specs + [_full((D, D)), _full((1, D)), _full((D, D)),
                              _full((1, D)), _full((REL, D, D))],
        out_specs=tbl_spec,
        out_shape=tbl_shape,
    )
    final = pl.pallas_call(
        _final_body,
        grid=(GRID,),
        in_specs=acc_specs + [_full((D, D)), _full((1, D))],
        out_specs=_row_spec(),
        out_shape=jax.ShapeDtypeStruct((N, D), f32),
    )
    return proj, prologue_tables, mid, final


def _fill(ref, rows, value):
    width = ref.shape[1]
    val = jnp.full((16,), value, dtype=ref.dtype)

    def body(r, _):
        for j in range(width // 16):
            ref[r, pl.ds(j * 16, 16)] = val
        return 0

    lax.fori_loop(0, rows, body, 0)


def _deg_body(col_hbm, deg_out, cstage, hist):
    c = lax.axis_index("c")
    s = lax.axis_index("s")
    w = c * NS + s

    def zero(r, _):
        hist[pl.ds(r * 16, 16)] = jnp.zeros((16,), jnp.float32)
        return 0

    lax.fori_loop(0, N // 16, zero, 0)
    one = jnp.ones((16,), jnp.float32)
    lane = lax.iota(jnp.int32, 16)
    base0 = w * DEG_PER_W

    def body(k, _):
        pltpu.sync_copy(col_hbm.at[pl.ds(base0 + k * SCH, SCH)], cstage)

        def grp(j, _):
            o = pl.multiple_of(j * 16, 16)
            c16 = cstage[pl.ds(o, 16)]
            for i in range(16):
                plsc.addupdate_scatter(hist, [c16], one, mask=lane == i)
            return 0

        lax.fori_loop(0, SCH // 16, grp, 0)
        return 0

    lax.fori_loop(0, DEG_PER_W // SCH, body, 0)
    pltpu.sync_copy(hist, deg_out.at[pl.ds(w * N, N)])


def _edge_body(tbl_hbm, row_hbm, col_hbm, typ_hbm, z128_hbm,
               outA, outB,
               rstage, cstage, tstage,
               lrow_b, gidx_b,
               row_v, gidx_v, gidx_v2, buf, buf2, rexbuf,
               accA, accB, sem1, sem2, sems1, sems2):
    c = lax.axis_index("c")
    s = lax.axis_index("s")
    goff = c * (REL * N)
    ebase = s * EPT
    _fill(rexbuf, CH, 0.0)

    for b in range(NB):
        lo = b * BKT

        def comp_chunk(k, p, lo=lo):
            base = ebase + k * SCH
            pltpu.sync_copy(row_hbm.at[pl.ds(base, SCH)], rstage)
            pltpu.sync_copy(col_hbm.at[pl.ds(base, SCH)], cstage)
            pltpu.sync_copy(typ_hbm.at[pl.ds(base, SCH)], tstage)

            def grp(j, p):
                o = pl.multiple_of(j * 16, 16)
                r16 = rstage[pl.ds(o, 16)] - lo
                c16 = cstage[pl.ds(o, 16)]
                t16 = tstage[pl.ds(o, 16)]
                g16 = t16 * N + c16 + goff
                m = jnp.logical_and(r16 >= 0, r16 < BKT)
                mi = jnp.where(m, 1, 0)
                incl = plsc.cumsum(mi)
                pos = p + (incl - mi)
                plsc.store_scatter(lrow_b, [pos], r16, mask=m)
                plsc.store_scatter(gidx_b, [pos], g16, mask=m)
                return p + jnp.max(incl)

            return lax.fori_loop(0, SCH // 16, grp, p)

        cnt = lax.fori_loop(0, NSCH, comp_chunk, 0)

        trash = jnp.full((16,), BKT, jnp.int32)
        gpad = jnp.full((16,), 0, jnp.int32) + goff
        for j in range(CH // 16):
            lrow_b[pl.ds(cnt + j * 16, 16)] = trash
            gidx_b[pl.ds(cnt + j * 16, 16)] = gpad

        pltpu.sync_copy(z128_hbm, accA.at[pl.ds(s * ZPT_A, ZPT_A)])
        pltpu.sync_copy(z128_hbm, accB.at[pl.ds(s * ZPT_A, ZPT_A)])
        plsc.subcore_barrier()

        nch = (cnt + CH - 1) // CH
        bufs = (buf, buf2)
        gidxs = (gidx_v, gidx_v2)
        gsems = (sem1, sem2)

        def fill_idx(kchunk, par):
            o = kchunk * CH

            def cp(j, _):
                oj = pl.multiple_of(j * 16, 16)
                gidxs[par][pl.ds(oj, 16)] = gidx_b[pl.ds(o + oj, 16)]
                return 0

            lax.fori_loop(0, CH // 16, cp, 0)
            pltpu.async_copy(tbl_hbm.at[gidxs[par]], bufs[par], gsems[par])

        @pl.when(nch > 0)
        def _():
            fill_idx(0, 0)

        def consume(k, par):
            o = k * CH

            def cp(j, _):
                oj = pl.multiple_of(j * 16, 16)
                row_v[pl.ds(oj, 16)] = lrow_b[pl.ds(o + oj, 16)]
                return 0

            lax.fori_loop(0, CH // 16, cp, 0)
            cbuf = bufs[par]
            pltpu.make_async_copy(tbl_hbm.at[gidxs[par]], cbuf,
                                  gsems[par]).wait()

            def exp_body(e, _):
                for j in range(4):
                    oj = pl.multiple_of(j * 16, 16)
                    z = cbuf[e, pl.ds(64 + oj, 16)]
                    ex = jnp.exp(z)
                    cbuf[e, pl.ds(64 + oj, 16)] = ex
                    rexbuf[e, pl.ds(oj, 16)] = z * ex
                return 0

            lax.fori_loop(0, CH, exp_body, 0)
            s1 = pltpu.async_copy(cbuf, accA.at[row_v], sems1, add=True)
            s2 = pltpu.async_copy(rexbuf, accB.at[row_v], sems2, add=True)
            s1.wait()
            s2.wait()

        def pair(kk, _):
            for par in range(2):
                k = 2 * kk + par

                @pl.when(k < nch)
                def _(k=k, par=par):
                    @pl.when(k + 1 < nch)
                    def _():
                        fill_idx(k + 1, 1 - par)

                    consume(k, par)

            return 0

        lax.fori_loop(0, (nch + 1) // 2, pair, 0)
        plsc.subcore_barrier()
        obase = (c * NB + b) * ACC_R + s * ZPT_A
        pltpu.sync_copy(accA.at[pl.ds(s * ZPT_A, ZPT_A)],
                        outA.at[pl.ds(obase, ZPT_A)])
        pltpu.sync_copy(accB.at[pl.ds(s * ZPT_A, ZPT_A)],
                        outB.at[pl.ds(obase, ZPT_A)])
        plsc.subcore_barrier()


def _gather_body(g_hbm, idx_hbm, out_hbm, idx_v, buf, sem):
    c = lax.axis_index("c")
    s = lax.axis_index("s")
    base = (c * NS + s) * (NIDX // (NC * NS))
    pltpu.sync_copy(idx_hbm.at[pl.ds(base, NIDX // (NC * NS))], idx_v)
    pltpu.async_copy(g_hbm.at[idx_v], buf, sem).wait()
    pltpu.sync_copy(buf, out_hbm.at[pl.ds(base, NIDX // (NC * NS))])


def _sc_calls():
    f32 = jnp.float32
    i32 = jnp.int32
    mesh = plsc.VectorSubcoreMesh(core_axis_name="c", subcore_axis_name="s",
                                  num_cores=NC, num_subcores=NS)
    deg = pl.kernel(
        _deg_body,
        out_type=jax.ShapeDtypeStruct((NC * NS * N,), f32),
        mesh=mesh,
        compiler_params=pltpu.CompilerParams(needs_layout_passes=False),
        scratch_types=[
            pltpu.VMEM((SCH,), i32),
            pltpu.VMEM((N,), f32),
        ],
    )
    edge = pl.kernel(
        _edge_body,
        out_type=(jax.ShapeDtypeStruct((NC * NB * ACC_R, D), f32),
                  jax.ShapeDtypeStruct((NC * NB * ACC_R, D), f32)),
        mesh=mesh,
        compiler_params=pltpu.CompilerParams(needs_layout_passes=False),
        scratch_types=[
            pltpu.VMEM((SCH,), i32), pltpu.VMEM((SCH,), i32),
            pltpu.VMEM((SCH,), i32),
            pltpu.VMEM((CAP,), i32), pltpu.VMEM((CAP,), i32),
            pltpu.VMEM((CH,), i32), pltpu.VMEM((CH,), i32),
            pltpu.VMEM((CH,), i32),
            pltpu.VMEM((CH, D), f32), pltpu.VMEM((CH, D), f32),
            pltpu.VMEM((CH, D), f32),
            pltpu.VMEM_SHARED((ACC_R, D), f32),
            pltpu.VMEM_SHARED((ACC_R, D), f32),
            pltpu.SemaphoreType.DMA, pltpu.SemaphoreType.DMA,
            pltpu.SemaphoreType.DMA, pltpu.SemaphoreType.DMA,
        ],
    )
    gather = pl.kernel(
        _gather_body,
        out_type=jax.ShapeDtypeStruct((NIDX, D), f32),
        mesh=mesh,
        scratch_types=[
            pltpu.VMEM((NIDX // (NC * NS),), i32),
            pltpu.VMEM((NIDX // (NC * NS), D), f32),
            pltpu.SemaphoreType.DMA,
        ],
    )
    return deg, edge, gather


def kernel(x, edge_index, idx, edge_type, edge_weight, proj_W, proj_b,
           bn_gamma, bn_beta, W_rel_0, Win_W_0, Win_b_0, Wout_W_0, Wout_b_0,
           W_rel_1, Win_W_1, Win_b_1, Wout_W_1, Wout_b_1):
    f32 = jnp.float32
    i32 = jnp.int32
    proj, prologue_tables, mid, final = _tc_calls()
    deg_k, edge_k, gather_k = _sc_calls()

    row = edge_index[0]
    col = edge_index[1]

    deg_out = deg_k(col)
    degp = deg_out.reshape(NC * NS, N).T

    h, stats = proj(x, proj_W, proj_b.reshape(1, D))
    z128 = jnp.zeros((ZPT_A, D), f32)

    def layer(tbl):
        aA, aB = edge_k(tbl.reshape(NC * REL * N, D), row, col, edge_type,
                        z128)

        def halves(a):
            def core(ci):
                o = ci * NB * ACC_R
                return jnp.concatenate(
                    [a[o + b * ACC_R:o + b * ACC_R + BKT] for b in range(NB)])

            return jnp.stack([core(0), core(1)])

        return halves(aA), halves(aB)

    tbl1 = prologue_tables(h, stats, bn_gamma.reshape(1, D),
                           bn_beta.reshape(1, D), Win_W_0,
                           Win_b_0.reshape(1, D), W_rel_0, degp)
    aA, aB = layer(tbl1)
    tbl2 = mid(aA, aB, degp, Wout_W_0, Wout_b_0.reshape(1, D),
               Win_W_1, Win_b_1.reshape(1, D), W_rel_1)
    aA, aB = layer(tbl2)
    g = final(aA, aB, degp, Wout_W_1, Wout_b_1.reshape(1, D))
    return gather_k(g, idx)


# speedup vs baseline: 9.3770x; 1.0717x over previous
import functools

import jax
import jax.numpy as jnp
from jax import lax
from jax.experimental import pallas as pl
from jax.experimental.pallas import tpu as pltpu
from jax.experimental.pallas import tpu_sc as plsc

N = 10000
E = 320000
D = 128
REL = 4
NIDX = 2048

NC = 2
NS = 16

CH = 128
EPT = 20000
NB = 5
BKT = N // NB
ACC_R = 2048
ZPT_A = ACC_R // NS
SCH = 2000
NSCH = EPT // SCH
CAP = 20224
ACC_ROWS = 10112
ZPT = ACC_ROWS // NS

DEG_CH = 80
DEG_PER_W = E // (NC * NS)
DEG_NCHUNK = DEG_PER_W // DEG_CH

BLK = 400
GRID = N // BLK


def _dis_from_deg(deg_blk):
    deg = jnp.sum(deg_blk, axis=1, keepdims=True)
    return jnp.where(deg > 0, lax.rsqrt(jnp.maximum(deg, 1e-12)), 0.0)


def _proj_body(x_ref, pw_ref, pb_ref, h_ref, st_ref, acc_ref):
    i = pl.program_id(0)
    h = lax.dot_general(x_ref[...], pw_ref[...], (((1,), (0,)), ((), ())),
                        preferred_element_type=jnp.float32) + pb_ref[...]
    h_ref[...] = h

    @pl.when(i == 0)
    def _():
        acc_ref[...] = jnp.zeros_like(acc_ref)

    acc_ref[0:1, :] += jnp.sum(h, axis=0, keepdims=True)
    acc_ref[1:2, :] += jnp.sum(h * h, axis=0, keepdims=True)

    @pl.when(i == GRID - 1)
    def _():
        st_ref[...] = acc_ref[...]


def _mm(a, b):
    return lax.dot_general(a, b, (((1,), (0,)), ((), ())),
                           preferred_element_type=jnp.float32)


def _tables_out(x1, dis, wrel_ref, tbl_ref):
    u = dis * x1
    for t in range(REL):
        z = _mm(x1, wrel_ref[t])
        tbl_ref[0, t] = jnp.concatenate([u[:, :64], z[:, :64]], axis=1)
        tbl_ref[1, t] = jnp.concatenate([u[:, 64:], z[:, 64:]], axis=1)


def _prologue_tables_body(h_ref, st_ref, g_ref, b_ref, win_ref, winb_ref,
                          wrel_ref, deg_ref, tbl_ref):
    mean = st_ref[0:1, :] / N
    var = st_ref[1:2, :] / N - mean * mean
    hn = (h_ref[...] - mean) * lax.rsqrt(var + 1e-5) * g_ref[...] + b_ref[...]
    cont = jnp.maximum(hn, 0.0)
    x1 = _mm(cont, win_ref[...]) + winb_ref[...]
    _tables_out(x1, _dis_from_deg(deg_ref[...]), wrel_ref, tbl_ref)


def _epilogue(aA_ref, aB_ref, deg_ref, wout_ref, woutb_ref):
    mg = jnp.concatenate([aA_ref[0, :, :64], aA_ref[1, :, :64]], axis=1)
    den = jnp.concatenate([aA_ref[0, :, 64:], aA_ref[1, :, 64:]], axis=1)
    num = jnp.concatenate([aB_ref[0, :, :64], aB_ref[1, :, :64]], axis=1)
    msg_gcn = _dis_from_deg(deg_ref[...]) * mg
    msg = num / (den + 1e-16)
    y = msg_gcn + 0.5 * jnp.maximum(msg, 0.0)
    return _mm(y, wout_ref[...]) + woutb_ref[...]


def _mid_body(aA_ref, aB_ref, deg_ref, wout_ref, woutb_ref,
              win_ref, winb_ref, wrel_ref, tbl_ref):
    cont = _epilogue(aA_ref, aB_ref, deg_ref, wout_ref, woutb_ref)
    x2 = _mm(cont, win_ref[...]) + winb_ref[...]
    _tables_out(x2, _dis_from_deg(deg_ref[...]), wrel_ref, tbl_ref)


def _erf(x):
    s = jnp.sign(x)
    a = jnp.abs(x)
    t = 1.0 / (1.0 + 0.3275911 * a)
    poly = t * (0.254829592 + t * (-0.284496736 + t * (1.421413741
               + t * (-1.453152027 + t * 1.061405429))))
    return s * (1.0 - poly * jnp.exp(-a * a))


def _final_body(aA_ref, aB_ref, deg_ref, wout_ref, woutb_ref, g_ref):
    cont = _epilogue(aA_ref, aB_ref, deg_ref, wout_ref, woutb_ref)
    g_ref[...] = cont * 0.5 * (1.0 + _erf(cont * 0.7071067811865476))


def _row_spec(shape_prefix=(), minor=D):
    nd = len(shape_prefix)
    return pl.BlockSpec(shape_prefix + (BLK, minor),
                        lambda i: (0,) * nd + (i, 0))


def _full(shape):
    return pl.BlockSpec(shape, lambda i: (0,) * len(shape))


def _tc_calls():
    f32 = jnp.float32
    proj = pl.pallas_call(
        _proj_body,
        grid=(GRID,),
        in_specs=[_row_spec(), _full((D, D)), _full((1, D))],
        out_specs=[_row_spec(), _full((8, D))],
        out_shape=[jax.ShapeDtypeStruct((N, D), f32),
                   jax.ShapeDtypeStruct((8, D), f32)],
        scratch_shapes=[pltpu.VMEM((8, D), f32)],
    )
    tbl_spec = pl.BlockSpec((2, REL, BLK, D), lambda i: (0, 0, i, 0))
    tbl_shape = jax.ShapeDtypeStruct((2, REL, N, D), f32)
    prologue_tables = pl.pallas_call(
        _prologue_tables_body,
        grid=(GRID,),
        in_specs=[_row_spec(), _full((8, D)), _full((1, D)), _full((1, D)),
                  _full((D, D)), _full((1, D)), _full((REL, D, D)),
                  _row_spec((), 32)],
        out_specs=tbl_spec,
        out_shape=tbl_shape,
    )
    acc_specs = [_row_spec((2,), D), _row_spec((2,), D), _row_spec((), 32)]
    mid = pl.pallas_call(
        _mid_body,
        grid=(GRID,),
        in_specs=acc_specs + [_full((D, D)), _full((1, D)), _full((D, D)),
                              _full((1, D)), _full((REL, D, D))],
        out_specs=tbl_spec,
        out_shape=tbl_shape,
    )
    final = pl.pallas_call(
        _final_body,
        grid=(GRID,),
        in_specs=acc_specs + [_full((D, D)), _full((1, D))],
        out_specs=_row_spec(),
        out_shape=jax.ShapeDtypeStruct((N, D), f32),
    )
    return proj, prologue_tables, mid, final


def _fill(ref, rows, value):
    width = ref.shape[1]
    val = jnp.full((16,), value, dtype=ref.dtype)

    def body(r, _):
        for j in range(width // 16):
            ref[r, pl.ds(j * 16, 16)] = val
        return 0

    lax.fori_loop(0, rows, body, 0)


def _deg_body(col_hbm, deg_out, cstage, hist):
    c = lax.axis_index("c")
    s = lax.axis_index("s")
    w = c * NS + s

    def zero(r, _):
        hist[pl.ds(r * 16, 16)] = jnp.zeros((16,), jnp.float32)
        return 0

    lax.fori_loop(0, N // 16, zero, 0)
    one = jnp.ones((16,), jnp.float32)
    lane = lax.iota(jnp.int32, 16)
    base0 = w * DEG_PER_W

    def body(k, _):
        pltpu.sync_copy(col_hbm.at[pl.ds(base0 + k * SCH, SCH)], cstage)

        def grp(j, _):
            o = pl.multiple_of(j * 16, 16)
            c16 = cstage[pl.ds(o, 16)]
            for i in range(16):
                plsc.addupdate_scatter(hist, [c16], one, mask=lane == i)
            return 0

        lax.fori_loop(0, SCH // 16, grp, 0)
        return 0

    lax.fori_loop(0, DEG_PER_W // SCH, body, 0)
    pltpu.sync_copy(hist, deg_out.at[pl.ds(w * N, N)])


def _edge_body(tbl_hbm, row_hbm, col_hbm, typ_hbm, z128_hbm,
               outA, outB,
               rstage, cstage, tstage,
               lrow_b, gidx_b,
               row_v, gidx_v, gidx_v2, buf, buf2, rexbuf,
               accA, accB, sem1, sem2, sems1, sems2):
    c = lax.axis_index("c")
    s = lax.axis_index("s")
    goff = c * (REL * N)
    ebase = s * EPT
    _fill(rexbuf, CH, 0.0)

    for b in range(NB):
        lo = b * BKT

        def comp_chunk(k, p, lo=lo):
            base = ebase + k * SCH
            pltpu.sync_copy(row_hbm.at[pl.ds(base, SCH)], rstage)
            pltpu.sync_copy(col_hbm.at[pl.ds(base, SCH)], cstage)
            pltpu.sync_copy(typ_hbm.at[pl.ds(base, SCH)], tstage)

            def grp(j, p):
                o = pl.multiple_of(j * 16, 16)
                r16 = rstage[pl.ds(o, 16)] - lo
                c16 = cstage[pl.ds(o, 16)]
                t16 = tstage[pl.ds(o, 16)]
                g16 = t16 * N + c16 + goff
                m = jnp.logical_and(r16 >= 0, r16 < BKT)
                mi = jnp.where(m, 1, 0)
                incl = plsc.cumsum(mi)
                pos = p + (incl - mi)
                plsc.store_scatter(lrow_b, [pos], r16, mask=m)
                plsc.store_scatter(gidx_b, [pos], g16, mask=m)
                return p + jnp.max(incl)

            return lax.fori_loop(0, SCH // 16, grp, p)

        cnt = lax.fori_loop(0, NSCH, comp_chunk, 0)

        trash = jnp.full((16,), BKT, jnp.int32)
        gpad = jnp.full((16,), 0, jnp.int32) + goff
        for j in range(CH // 16):
            lrow_b[pl.ds(cnt + j * 16, 16)] = trash
            gidx_b[pl.ds(cnt + j * 16, 16)] = gpad

        pltpu.sync_copy(z128_hbm, accA.at[pl.ds(s * ZPT_A, ZPT_A)])
        pltpu.sync_copy(z128_hbm, accB.at[pl.ds(s * ZPT_A, ZPT_A)])
        plsc.subcore_barrier()

        nch = (cnt + CH - 1) // CH
        bufs = (buf, buf2)
        gidxs = (gidx_v, gidx_v2)
        gsems = (sem1, sem2)

        def fill_idx(kchunk, par):
            o = kchunk * CH

            def cp(j, _):
                oj = pl.multiple_of(j * 16, 16)
                gidxs[par][pl.ds(oj, 16)] = gidx_b[pl.ds(o + oj, 16)]
                return 0

            lax.fori_loop(0, CH // 16, cp, 0)
            pltpu.async_copy(tbl_hbm.at[gidxs[par]], bufs[par], gsems[par])

        @pl.when(nch > 0)
        def _():
            fill_idx(0, 0)

        def consume(k, par):
            o = k * CH

            def cp(j, _):
                oj = pl.multiple_of(j * 16, 16)
                row_v[pl.ds(oj, 16)] = lrow_b[pl.ds(o + oj, 16)]
                return 0

            lax.fori_loop(0, CH // 16, cp, 0)
            cbuf = bufs[par]
            pltpu.make_async_copy(tbl_hbm.at[gidxs[par]], cbuf,
                                  gsems[par]).wait()

            @plsc.parallel_loop(0, CH, unroll=4)
            def exp_body(e):
                for j in range(4):
                    oj = pl.multiple_of(j * 16, 16)
                    z = cbuf[e, pl.ds(64 + oj, 16)]
                    ex = jnp.exp(z)
                    cbuf[e, pl.ds(64 + oj, 16)] = ex
                    rexbuf[e, pl.ds(oj, 16)] = z * ex
            s1 = pltpu.async_copy(cbuf, accA.at[row_v], sems1, add=True)
            s2 = pltpu.async_copy(rexbuf, accB.at[row_v], sems2, add=True)
            s1.wait()
            s2.wait()

        def pair(kk, _):
            for par in range(2):
                k = 2 * kk + par

                @pl.when(k < nch)
                def _(k=k, par=par):
                    @pl.when(k + 1 < nch)
                    def _():
                        fill_idx(k + 1, 1 - par)

                    consume(k, par)

            return 0

        lax.fori_loop(0, (nch + 1) // 2, pair, 0)
        plsc.subcore_barrier()
        obase = (c * NB + b) * ACC_R + s * ZPT_A
        pltpu.sync_copy(accA.at[pl.ds(s * ZPT_A, ZPT_A)],
                        outA.at[pl.ds(obase, ZPT_A)])
        pltpu.sync_copy(accB.at[pl.ds(s * ZPT_A, ZPT_A)],
                        outB.at[pl.ds(obase, ZPT_A)])
        plsc.subcore_barrier()


def _gather_body(g_hbm, idx_hbm, out_hbm, idx_v, buf, sem):
    c = lax.axis_index("c")
    s = lax.axis_index("s")
    base = (c * NS + s) * (NIDX // (NC * NS))
    pltpu.sync_copy(idx_hbm.at[pl.ds(base, NIDX // (NC * NS))], idx_v)
    pltpu.async_copy(g_hbm.at[idx_v], buf, sem).wait()
    pltpu.sync_copy(buf, out_hbm.at[pl.ds(base, NIDX // (NC * NS))])


def _sc_calls():
    f32 = jnp.float32
    i32 = jnp.int32
    mesh = plsc.VectorSubcoreMesh(core_axis_name="c", subcore_axis_name="s",
                                  num_cores=NC, num_subcores=NS)
    deg = pl.kernel(
        _deg_body,
        out_type=jax.ShapeDtypeStruct((NC * NS * N,), f32),
        mesh=mesh,
        compiler_params=pltpu.CompilerParams(needs_layout_passes=False),
        scratch_types=[
            pltpu.VMEM((SCH,), i32),
            pltpu.VMEM((N,), f32),
        ],
    )
    edge = pl.kernel(
        _edge_body,
        out_type=(jax.ShapeDtypeStruct((NC * NB * ACC_R, D), f32),
                  jax.ShapeDtypeStruct((NC * NB * ACC_R, D), f32)),
        mesh=mesh,
        compiler_params=pltpu.CompilerParams(needs_layout_passes=False),
        scratch_types=[
            pltpu.VMEM((SCH,), i32), pltpu.VMEM((SCH,), i32),
            pltpu.VMEM((SCH,), i32),
            pltpu.VMEM((CAP,), i32), pltpu.VMEM((CAP,), i32),
            pltpu.VMEM((CH,), i32), pltpu.VMEM((CH,), i32),
            pltpu.VMEM((CH,), i32),
            pltpu.VMEM((CH, D), f32), pltpu.VMEM((CH, D), f32),
            pltpu.VMEM((CH, D), f32),
            pltpu.VMEM_SHARED((ACC_R, D), f32),
            pltpu.VMEM_SHARED((ACC_R, D), f32),
            pltpu.SemaphoreType.DMA, pltpu.SemaphoreType.DMA,
            pltpu.SemaphoreType.DMA, pltpu.SemaphoreType.DMA,
        ],
    )
    gather = pl.kernel(
        _gather_body,
        out_type=jax.ShapeDtypeStruct((NIDX, D), f32),
        mesh=mesh,
        scratch_types=[
            pltpu.VMEM((NIDX // (NC * NS),), i32),
            pltpu.VMEM((NIDX // (NC * NS), D), f32),
            pltpu.SemaphoreType.DMA,
        ],
    )
    return deg, edge, gather


def kernel(x, edge_index, idx, edge_type, edge_weight, proj_W, proj_b,
           bn_gamma, bn_beta, W_rel_0, Win_W_0, Win_b_0, Wout_W_0, Wout_b_0,
           W_rel_1, Win_W_1, Win_b_1, Wout_W_1, Wout_b_1):
    f32 = jnp.float32
    i32 = jnp.int32
    proj, prologue_tables, mid, final = _tc_calls()
    deg_k, edge_k, gather_k = _sc_calls()

    row = edge_index[0]
    col = edge_index[1]

    deg_out = deg_k(col)
    degp = deg_out.reshape(NC * NS, N).T

    h, stats = proj(x, proj_W, proj_b.reshape(1, D))
    z128 = jnp.zeros((ZPT_A, D), f32)

    def layer(tbl):
        aA, aB = edge_k(tbl.reshape(NC * REL * N, D), row, col, edge_type,
                        z128)

        def halves(a):
            def core(ci):
                o = ci * NB * ACC_R
                return jnp.concatenate(
                    [a[o + b * ACC_R:o + b * ACC_R + BKT] for b in range(NB)])

            return jnp.stack([core(0), core(1)])

        return halves(aA), halves(aB)

    tbl1 = prologue_tables(h, stats, bn_gamma.reshape(1, D),
                           bn_beta.reshape(1, D), Win_W_0,
                           Win_b_0.reshape(1, D), W_rel_0, degp)
    aA, aB = layer(tbl1)
    tbl2 = mid(aA, aB, degp, Wout_W_0, Wout_b_0.reshape(1, D),
               Win_W_1, Win_b_1.reshape(1, D), W_rel_1)
    aA, aB = layer(tbl2)
    g = final(aA, aB, degp, Wout_W_1, Wout_b_1.reshape(1, D))
    return gather_k(g, idx)


# speedup vs baseline: 10.1307x; 1.0804x over previous
import functools

import jax
import jax.numpy as jnp
from jax import lax
from jax.experimental import pallas as pl
from jax.experimental.pallas import tpu as pltpu
from jax.experimental.pallas import tpu_sc as plsc

N = 10000
E = 320000
D = 128
REL = 4
NIDX = 2048

NC = 2
NS = 16

CH = 128
EPT = 20000
NB = 5
BKT = N // NB
ACC_R = 2048
ZPT_A = ACC_R // NS
SCH = 2000
NSCH = EPT // SCH
CAP = 20224
ACC_ROWS = 10112
ZPT = ACC_ROWS // NS

DEG_CH = 80
DEG_PER_W = E // (NC * NS)
DEG_NCHUNK = DEG_PER_W // DEG_CH

BLK = 400
GRID = N // BLK


def _dis_from_deg(deg_blk):
    deg = jnp.sum(deg_blk, axis=1, keepdims=True)
    return jnp.where(deg > 0, lax.rsqrt(jnp.maximum(deg, 1e-12)), 0.0)


def _proj_body(x_ref, pw_ref, pb_ref, h_ref, st_ref, acc_ref):
    i = pl.program_id(0)
    h = lax.dot_general(x_ref[...], pw_ref[...], (((1,), (0,)), ((), ())),
                        preferred_element_type=jnp.float32) + pb_ref[...]
    h_ref[...] = h

    @pl.when(i == 0)
    def _():
        acc_ref[...] = jnp.zeros_like(acc_ref)

    acc_ref[0:1, :] += jnp.sum(h, axis=0, keepdims=True)
    acc_ref[1:2, :] += jnp.sum(h * h, axis=0, keepdims=True)

    @pl.when(i == GRID - 1)
    def _():
        st_ref[...] = acc_ref[...]


def _mm(a, b):
    return lax.dot_general(a, b, (((1,), (0,)), ((), ())),
                           preferred_element_type=jnp.float32)


def _tables_out(x1, dis, wrel_ref, tbl_ref):
    u = dis * x1
    for t in range(REL):
        z = _mm(x1, wrel_ref[t])
        tbl_ref[0, t] = jnp.concatenate([u[:, :64], z[:, :64]], axis=1)
        tbl_ref[1, t] = jnp.concatenate([u[:, 64:], z[:, 64:]], axis=1)


def _prologue_tables_body(h_ref, st_ref, g_ref, b_ref, win_ref, winb_ref,
                          wrel_ref, deg_ref, tbl_ref):
    mean = st_ref[0:1, :] / N
    var = st_ref[1:2, :] / N - mean * mean
    hn = (h_ref[...] - mean) * lax.rsqrt(var + 1e-5) * g_ref[...] + b_ref[...]
    cont = jnp.maximum(hn, 0.0)
    x1 = _mm(cont, win_ref[...]) + winb_ref[...]
    _tables_out(x1, _dis_from_deg(deg_ref[...]), wrel_ref, tbl_ref)


def _epilogue(aA_ref, aB_ref, deg_ref, wout_ref, woutb_ref):
    mg = jnp.concatenate([aA_ref[0, :, :64], aA_ref[1, :, :64]], axis=1)
    den = jnp.concatenate([aA_ref[0, :, 64:], aA_ref[1, :, 64:]], axis=1)
    num = jnp.concatenate([aB_ref[0, :, :64], aB_ref[1, :, :64]], axis=1)
    msg_gcn = _dis_from_deg(deg_ref[...]) * mg
    msg = num / (den + 1e-16)
    y = msg_gcn + 0.5 * jnp.maximum(msg, 0.0)
    return _mm(y, wout_ref[...]) + woutb_ref[...]


def _mid_body(aA_ref, aB_ref, deg_ref, wout_ref, woutb_ref,
              win_ref, winb_ref, wrel_ref, tbl_ref):
    cont = _epilogue(aA_ref, aB_ref, deg_ref, wout_ref, woutb_ref)
    x2 = _mm(cont, win_ref[...]) + winb_ref[...]
    _tables_out(x2, _dis_from_deg(deg_ref[...]), wrel_ref, tbl_ref)


def _erf(x):
    s = jnp.sign(x)
    a = jnp.abs(x)
    t = 1.0 / (1.0 + 0.3275911 * a)
    poly = t * (0.254829592 + t * (-0.284496736 + t * (1.421413741
               + t * (-1.453152027 + t * 1.061405429))))
    return s * (1.0 - poly * jnp.exp(-a * a))


def _final_body(aA_ref, aB_ref, deg_ref, wout_ref, woutb_ref, g_ref):
    cont = _epilogue(aA_ref, aB_ref, deg_ref, wout_ref, woutb_ref)
    g_ref[...] = cont * 0.5 * (1.0 + _erf(cont * 0.7071067811865476))


def _row_spec(shape_prefix=(), minor=D):
    nd = len(shape_prefix)
    return pl.BlockSpec(shape_prefix + (BLK, minor),
                        lambda i: (0,) * nd + (i, 0))


def _full(shape):
    return pl.BlockSpec(shape, lambda i: (0,) * len(shape))


def _tc_calls():
    f32 = jnp.float32
    proj = pl.pallas_call(
        _proj_body,
        grid=(GRID,),
        in_specs=[_row_spec(), _full((D, D)), _full((1, D))],
        out_specs=[_row_spec(), _full((8, D))],
        out_shape=[jax.ShapeDtypeStruct((N, D), f32),
                   jax.ShapeDtypeStruct((8, D), f32)],
        scratch_shapes=[pltpu.VMEM((8, D), f32)],
    )
    tbl_spec = pl.BlockSpec((2, REL, BLK, D), lambda i: (0, 0, i, 0))
    tbl_shape = jax.ShapeDtypeStruct((2, REL, N, D), f32)
    prologue_tables = pl.pallas_call(
        _prologue_tables_body,
        grid=(GRID,),
        in_specs=[_row_spec(), _full((8, D)), _full((1, D)), _full((1, D)),
                  _full((D, D)), _full((1, D)), _full((REL, D, D)),
                  _row_spec((), 32)],
        out_specs=tbl_spec,
        out_shape=tbl_shape,
    )
    acc_specs = [_row_spec((2,), D), _row_spec((2,), D), _row_spec((), 32)]
    mid = pl.pallas_call(
        _mid_body,
        grid=(GRID,),
        in_specs=acc_specs + [_full((D, D)), _full((1, D)), _full((D, D)),
                              _full((1, D)), _full((REL, D, D))],
        out_specs=tbl_spec,
        out_shape=tbl_shape,
    )
    final = pl.pallas_call(
        _final_body,
        grid=(GRID,),
        in_specs=acc_specs + [_full((D, D)), _full((1, D))],
        out_specs=_row_spec(),
        out_shape=jax.ShapeDtypeStruct((N, D), f32),
    )
    return proj, prologue_tables, mid, final


def _fill(ref, rows, value):
    width = ref.shape[1]
    val = jnp.full((16,), value, dtype=ref.dtype)

    def body(r, _):
        for j in range(width // 16):
            ref[r, pl.ds(j * 16, 16)] = val
        return 0

    lax.fori_loop(0, rows, body, 0)


def _deg_body(col_hbm, deg_out, cstage, hist):
    c = lax.axis_index("c")
    s = lax.axis_index("s")
    w = c * NS + s

    def zero(r, _):
        hist[pl.ds(r * 16, 16)] = jnp.zeros((16,), jnp.float32)
        return 0

    lax.fori_loop(0, N // 16, zero, 0)
    one = jnp.ones((16,), jnp.float32)
    lane = lax.iota(jnp.int32, 16)
    base0 = w * DEG_PER_W

    def body(k, _):
        pltpu.sync_copy(col_hbm.at[pl.ds(base0 + k * SCH, SCH)], cstage)

        def grp(j, _):
            o = pl.multiple_of(j * 16, 16)
            c16 = cstage[pl.ds(o, 16)]
            for i in range(16):
                plsc.addupdate_scatter(hist, [c16], one, mask=lane == i)
            return 0

        lax.fori_loop(0, SCH // 16, grp, 0)
        return 0

    lax.fori_loop(0, DEG_PER_W // SCH, body, 0)
    pltpu.sync_copy(hist, deg_out.at[pl.ds(w * N, N)])


def _edge_body(tbl_hbm, row_hbm, col_hbm, typ_hbm, z128_hbm,
               outA, outB,
               rstage, cstage, tstage,
               lrow_b, gidx_b,
               row_v, gidx_v, gidx_v2, buf, buf2, rexbuf,
               accA, accB, sem1, sem2, sems1, sems2):
    c = lax.axis_index("c")
    s = lax.axis_index("s")
    goff = c * (REL * N)
    ebase = s * EPT
    _fill(rexbuf, CH, 0.0)

    for b in range(NB):
        lo = b * BKT

        def comp_chunk(k, p, lo=lo):
            base = ebase + k * SCH
            pltpu.sync_copy(row_hbm.at[pl.ds(base, SCH)], rstage)
            pltpu.sync_copy(col_hbm.at[pl.ds(base, SCH)], cstage)
            pltpu.sync_copy(typ_hbm.at[pl.ds(base, SCH)], tstage)

            @plsc.parallel_loop(0, SCH // 16, unroll=4, carry=p)
            def grp(j, p):
                o = pl.multiple_of(j * 16, 16)
                r16 = rstage[pl.ds(o, 16)] - lo
                c16 = cstage[pl.ds(o, 16)]
                t16 = tstage[pl.ds(o, 16)]
                g16 = t16 * N + c16 + goff
                m = jnp.logical_and(r16 >= 0, r16 < BKT)
                mi = jnp.where(m, 1, 0)
                incl = plsc.cumsum(mi)
                pos = p + (incl - mi)
                plsc.store_scatter(lrow_b, [pos], r16, mask=m)
                plsc.store_scatter(gidx_b, [pos], g16, mask=m)
                return p + jnp.max(incl)

            return grp

        cnt = lax.fori_loop(0, NSCH, comp_chunk, 0)

        trash = jnp.full((16,), BKT, jnp.int32)
        gpad = jnp.full((16,), 0, jnp.int32) + goff
        for j in range(CH // 16):
            lrow_b[pl.ds(cnt + j * 16, 16)] = trash
            gidx_b[pl.ds(cnt + j * 16, 16)] = gpad

        pltpu.sync_copy(z128_hbm, accA.at[pl.ds(s * ZPT_A, ZPT_A)])
        pltpu.sync_copy(z128_hbm, accB.at[pl.ds(s * ZPT_A, ZPT_A)])
        plsc.subcore_barrier()

        nch = (cnt + CH - 1) // CH
        bufs = (buf, buf2)
        gidxs = (gidx_v, gidx_v2)
        gsems = (sem1, sem2)

        def fill_idx(kchunk, par):
            o = kchunk * CH

            def cp(j, _):
                oj = pl.multiple_of(j * 16, 16)
                gidxs[par][pl.ds(oj, 16)] = gidx_b[pl.ds(o + oj, 16)]
                return 0

            lax.fori_loop(0, CH // 16, cp, 0)
            pltpu.async_copy(tbl_hbm.at[gidxs[par]], bufs[par], gsems[par])

        @pl.when(nch > 0)
        def _():
            fill_idx(0, 0)

        def consume(k, par):
            o = k * CH

            def cp(j, _):
                oj = pl.multiple_of(j * 16, 16)
                row_v[pl.ds(oj, 16)] = lrow_b[pl.ds(o + oj, 16)]
                return 0

            lax.fori_loop(0, CH // 16, cp, 0)
            cbuf = bufs[par]
            pltpu.make_async_copy(tbl_hbm.at[gidxs[par]], cbuf,
                                  gsems[par]).wait()

            @plsc.parallel_loop(0, CH, unroll=4)
            def exp_body(e):
                for j in range(4):
                    oj = pl.multiple_of(j * 16, 16)
                    z = cbuf[e, pl.ds(64 + oj, 16)]
                    ex = jnp.exp(z)
                    cbuf[e, pl.ds(64 + oj, 16)] = ex
                    rexbuf[e, pl.ds(oj, 16)] = z * ex
            s1 = pltpu.async_copy(cbuf, accA.at[row_v], sems1, add=True)
            s2 = pltpu.async_copy(rexbuf, accB.at[row_v], sems2, add=True)
            s1.wait()
            s2.wait()

        def pair(kk, _):
            for par in range(2):
                k = 2 * kk + par

                @pl.when(k < nch)
                def _(k=k, par=par):
                    @pl.when(k + 1 < nch)
                    def _():
                        fill_idx(k + 1, 1 - par)

                    consume(k, par)

            return 0

        lax.fori_loop(0, (nch + 1) // 2, pair, 0)
        plsc.subcore_barrier()
        obase = (c * NB + b) * ACC_R + s * ZPT_A
        pltpu.sync_copy(accA.at[pl.ds(s * ZPT_A, ZPT_A)],
                        outA.at[pl.ds(obase, ZPT_A)])
        pltpu.sync_copy(accB.at[pl.ds(s * ZPT_A, ZPT_A)],
                        outB.at[pl.ds(obase, ZPT_A)])
        plsc.subcore_barrier()


def _gather_body(g_hbm, idx_hbm, out_hbm, idx_v, buf, sem):
    c = lax.axis_index("c")
    s = lax.axis_index("s")
    base = (c * NS + s) * (NIDX // (NC * NS))
    pltpu.sync_copy(idx_hbm.at[pl.ds(base, NIDX // (NC * NS))], idx_v)
    pltpu.async_copy(g_hbm.at[idx_v], buf, sem).wait()
    pltpu.sync_copy(buf, out_hbm.at[pl.ds(base, NIDX // (NC * NS))])


def _sc_calls():
    f32 = jnp.float32
    i32 = jnp.int32
    mesh = plsc.VectorSubcoreMesh(core_axis_name="c", subcore_axis_name="s",
                                  num_cores=NC, num_subcores=NS)
    deg = pl.kernel(
        _deg_body,
        out_type=jax.ShapeDtypeStruct((NC * NS * N,), f32),
        mesh=mesh,
        compiler_params=pltpu.CompilerParams(needs_layout_passes=False),
        scratch_types=[
            pltpu.VMEM((SCH,), i32),
            pltpu.VMEM((N,), f32),
        ],
    )
    edge = pl.kernel(
        _edge_body,
        out_type=(jax.ShapeDtypeStruct((NC * NB * ACC_R, D), f32),
                  jax.ShapeDtypeStruct((NC * NB * ACC_R, D), f32)),
        mesh=mesh,
        compiler_params=pltpu.CompilerParams(needs_layout_passes=False),
        scratch_types=[
            pltpu.VMEM((SCH,), i32), pltpu.VMEM((SCH,), i32),
            pltpu.VMEM((SCH,), i32),
            pltpu.VMEM((CAP,), i32), pltpu.VMEM((CAP,), i32),
            pltpu.VMEM((CH,), i32), pltpu.VMEM((CH,), i32),
            pltpu.VMEM((CH,), i32),
            pltpu.VMEM((CH, D), f32), pltpu.VMEM((CH, D), f32),
            pltpu.VMEM((CH, D), f32),
            pltpu.VMEM_SHARED((ACC_R, D), f32),
            pltpu.VMEM_SHARED((ACC_R, D), f32),
            pltpu.SemaphoreType.DMA, pltpu.SemaphoreType.DMA,
            pltpu.SemaphoreType.DMA, pltpu.SemaphoreType.DMA,
        ],
    )
    gather = pl.kernel(
        _gather_body,
        out_type=jax.ShapeDtypeStruct((NIDX, D), f32),
        mesh=mesh,
        scratch_types=[
            pltpu.VMEM((NIDX // (NC * NS),), i32),
            pltpu.VMEM((NIDX // (NC * NS), D), f32),
            pltpu.SemaphoreType.DMA,
        ],
    )
    return deg, edge, gather


def kernel(x, edge_index, idx, edge_type, edge_weight, proj_W, proj_b,
           bn_gamma, bn_beta, W_rel_0, Win_W_0, Win_b_0, Wout_W_0, Wout_b_0,
           W_rel_1, Win_W_1, Win_b_1, Wout_W_1, Wout_b_1):
    f32 = jnp.float32
    i32 = jnp.int32
    proj, prologue_tables, mid, final = _tc_calls()
    deg_k, edge_k, gather_k = _sc_calls()

    row = edge_index[0]
    col = edge_index[1]

    deg_out = deg_k(col)
    degp = deg_out.reshape(NC * NS, N).T

    h, stats = proj(x, proj_W, proj_b.reshape(1, D))
    z128 = jnp.zeros((ZPT_A, D), f32)

    def layer(tbl):
        aA, aB = edge_k(tbl.reshape(NC * REL * N, D), row, col, edge_type,
                        z128)

        def halves(a):
            def core(ci):
                o = ci * NB * ACC_R
                return jnp.concatenate(
                    [a[o + b * ACC_R:o + b * ACC_R + BKT] for b in range(NB)])

            return jnp.stack([core(0), core(1)])

        return halves(aA), halves(aB)

    tbl1 = prologue_tables(h, stats, bn_gamma.reshape(1, D),
                           bn_beta.reshape(1, D), Win_W_0,
                           Win_b_0.reshape(1, D), W_rel_0, degp)
    aA, aB = layer(tbl1)
    tbl2 = mid(aA, aB, degp, Wout_W_0, Wout_b_0.reshape(1, D),
               Win_W_1, Win_b_1.reshape(1, D), W_rel_1)
    aA, aB = layer(tbl2)
    g = final(aA, aB, degp, Wout_W_1, Wout_b_1.reshape(1, D))
    return gather_k(g, idx)


# speedup vs baseline: 10.2134x; 1.0082x over previous
import functools

import jax
import jax.numpy as jnp
from jax import lax
from jax.experimental import pallas as pl
from jax.experimental.pallas import tpu as pltpu
from jax.experimental.pallas import tpu_sc as plsc

N = 10000
E = 320000
D = 128
REL = 4
NIDX = 2048

NC = 2
NS = 16

CH = 128
EPT = 20000
NB = 5
BKT = N // NB
ACC_R = 2048
ZPT_A = ACC_R // NS
SCH = 2000
NSCH = EPT // SCH
CAP = 20224
ACC_ROWS = 10112
ZPT = ACC_ROWS // NS

DEG_CH = 80
DEG_PER_W = E // (NC * NS)
DEG_NCHUNK = DEG_PER_W // DEG_CH

BLK = 400
GRID = N // BLK


def _dis_from_deg(deg_blk):
    deg = jnp.sum(deg_blk, axis=1, keepdims=True)
    return jnp.where(deg > 0, lax.rsqrt(jnp.maximum(deg, 1e-12)), 0.0)


def _proj_body(x_ref, pw_ref, pb_ref, h_ref, st_ref, acc_ref):
    i = pl.program_id(0)
    h = lax.dot_general(x_ref[...], pw_ref[...], (((1,), (0,)), ((), ())),
                        preferred_element_type=jnp.float32) + pb_ref[...]
    h_ref[...] = h

    @pl.when(i == 0)
    def _():
        acc_ref[...] = jnp.zeros_like(acc_ref)

    acc_ref[0:1, :] += jnp.sum(h, axis=0, keepdims=True)
    acc_ref[1:2, :] += jnp.sum(h * h, axis=0, keepdims=True)

    @pl.when(i == GRID - 1)
    def _():
        st_ref[...] = acc_ref[...]


def _mm(a, b):
    return lax.dot_general(a, b, (((1,), (0,)), ((), ())),
                           preferred_element_type=jnp.float32)


def _tables_out(x1, dis, wrel_ref, tbl_ref):
    u = dis * x1
    for t in range(REL):
        z = _mm(x1, wrel_ref[t])
        tbl_ref[0, t] = jnp.concatenate([u[:, :64], z[:, :64]], axis=1)
        tbl_ref[1, t] = jnp.concatenate([u[:, 64:], z[:, 64:]], axis=1)


def _prologue_tables_body(h_ref, st_ref, g_ref, b_ref, win_ref, winb_ref,
                          wrel_ref, deg_ref, tbl_ref):
    mean = st_ref[0:1, :] / N
    var = st_ref[1:2, :] / N - mean * mean
    hn = (h_ref[...] - mean) * lax.rsqrt(var + 1e-5) * g_ref[...] + b_ref[...]
    cont = jnp.maximum(hn, 0.0)
    x1 = _mm(cont, win_ref[...]) + winb_ref[...]
    _tables_out(x1, _dis_from_deg(deg_ref[...]), wrel_ref, tbl_ref)


def _epilogue(aA_ref, aB_ref, deg_ref, wout_ref, woutb_ref):
    mg = jnp.concatenate([aA_ref[0, :, :64], aA_ref[1, :, :64]], axis=1)
    den = jnp.concatenate([aA_ref[0, :, 64:], aA_ref[1, :, 64:]], axis=1)
    num = jnp.concatenate([aB_ref[0, :, :64], aB_ref[1, :, :64]], axis=1)
    msg_gcn = _dis_from_deg(deg_ref[...]) * mg
    msg = num / (den + 1e-16)
    y = msg_gcn + 0.5 * jnp.maximum(msg, 0.0)
    return _mm(y, wout_ref[...]) + woutb_ref[...]


def _mid_body(aA_ref, aB_ref, deg_ref, wout_ref, woutb_ref,
              win_ref, winb_ref, wrel_ref, tbl_ref):
    cont = _epilogue(aA_ref, aB_ref, deg_ref, wout_ref, woutb_ref)
    x2 = _mm(cont, win_ref[...]) + winb_ref[...]
    _tables_out(x2, _dis_from_deg(deg_ref[...]), wrel_ref, tbl_ref)


def _erf(x):
    s = jnp.sign(x)
    a = jnp.abs(x)
    t = 1.0 / (1.0 + 0.3275911 * a)
    poly = t * (0.254829592 + t * (-0.284496736 + t * (1.421413741
               + t * (-1.453152027 + t * 1.061405429))))
    return s * (1.0 - poly * jnp.exp(-a * a))


def _final_body(aA_ref, aB_ref, deg_ref, wout_ref, woutb_ref, g_ref):
    cont = _epilogue(aA_ref, aB_ref, deg_ref, wout_ref, woutb_ref)
    g_ref[...] = cont * 0.5 * (1.0 + _erf(cont * 0.7071067811865476))


def _row_spec(shape_prefix=(), minor=D):
    nd = len(shape_prefix)
    return pl.BlockSpec(shape_prefix + (BLK, minor),
                        lambda i: (0,) * nd + (i, 0))


def _full(shape):
    return pl.BlockSpec(shape, lambda i: (0,) * len(shape))


def _tc_calls():
    f32 = jnp.float32
    proj = pl.pallas_call(
        _proj_body,
        grid=(GRID,),
        in_specs=[_row_spec(), _full((D, D)), _full((1, D))],
        out_specs=[_row_spec(), _full((8, D))],
        out_shape=[jax.ShapeDtypeStruct((N, D), f32),
                   jax.ShapeDtypeStruct((8, D), f32)],
        scratch_shapes=[pltpu.VMEM((8, D), f32)],
    )
    tbl_spec = pl.BlockSpec((2, REL, BLK, D), lambda i: (0, 0, i, 0))
    tbl_shape = jax.ShapeDtypeStruct((2, REL, N, D), f32)
    prologue_tables = pl.pallas_call(
        _prologue_tables_body,
        grid=(GRID,),
        in_specs=[_row_spec(), _full((8, D)), _full((1, D)), _full((1, D)),
                  _full((D, D)), _full((1, D)), _full((REL, D, D)),
                  _row_spec((), 32)],
        out_specs=tbl_spec,
        out_shape=tbl_shape,
    )
    acc_specs = [_row_spec((2,), D), _row_spec((2,), D), _row_spec((), 32)]
    mid = pl.pallas_call(
        _mid_body,
        grid=(GRID,),
        in_specs=acc_specs + [_full((D, D)), _full((1, D)), _full((D, D)),
                              _full((1, D)), _full((REL, D, D))],
        out_specs=tbl_spec,
        out_shape=tbl_shape,
    )
    final = pl.pallas_call(
        _final_body,
        grid=(GRID,),
        in_specs=acc_specs + [_full((D, D)), _full((1, D))],
        out_specs=_row_spec(),
        out_shape=jax.ShapeDtypeStruct((N, D), f32),
    )
    return proj, prologue_tables, mid, final


def _fill(ref, rows, value):
    width = ref.shape[1]
    val = jnp.full((16,), value, dtype=ref.dtype)

    def body(r, _):
        for j in range(width // 16):
            ref[r, pl.ds(j * 16, 16)] = val
        return 0

    lax.fori_loop(0, rows, body, 0)


def _deg_body(col_hbm, deg_out, cstage, hist):
    c = lax.axis_index("c")
    s = lax.axis_index("s")
    w = c * NS + s

    def zero(r, _):
        hist[pl.ds(r * 16, 16)] = jnp.zeros((16,), jnp.float32)
        return 0

    lax.fori_loop(0, N // 16, zero, 0)
    one = jnp.ones((16,), jnp.float32)
    lane = lax.iota(jnp.int32, 16)
    base0 = w * DEG_PER_W

    def body(k, _):
        pltpu.sync_copy(col_hbm.at[pl.ds(base0 + k * SCH, SCH)], cstage)

        def grp(j, _):
            o = pl.multiple_of(j * 16, 16)
            c16 = cstage[pl.ds(o, 16)]
            for i in range(16):
                plsc.addupdate_scatter(hist, [c16], one, mask=lane == i)
            return 0

        lax.fori_loop(0, SCH // 16, grp, 0)
        return 0

    lax.fori_loop(0, DEG_PER_W // SCH, body, 0)
    pltpu.sync_copy(hist, deg_out.at[pl.ds(w * N, N)])


def _edge_body(tbl_hbm, row_hbm, col_hbm, typ_hbm, z128_hbm,
               outA, outB,
               rstage, cstage, tstage,
               lrow_b, gidx_b,
               row_v, gidx_v, gidx_v2, buf, buf2, rexbuf,
               accA, accB, sem1, sem2, sems1, sems2):
    c = lax.axis_index("c")
    s = lax.axis_index("s")
    goff = c * (REL * N)
    ebase = s * EPT
    _fill(rexbuf, CH, 0.0)

    for b in range(NB):
        lo = b * BKT

        def comp_chunk(k, p, lo=lo):
            base = ebase + k * SCH
            pltpu.sync_copy(row_hbm.at[pl.ds(base, SCH)], rstage)
            pltpu.sync_copy(col_hbm.at[pl.ds(base, SCH)], cstage)
            pltpu.sync_copy(typ_hbm.at[pl.ds(base, SCH)], tstage)

            @plsc.parallel_loop(0, SCH // 16, unroll=4, carry=p)
            def grp(j, p):
                o = pl.multiple_of(j * 16, 16)
                r16 = rstage[pl.ds(o, 16)] - lo
                c16 = cstage[pl.ds(o, 16)]
                t16 = tstage[pl.ds(o, 16)]
                g16 = t16 * N + c16 + goff
                m = jnp.logical_and(r16 >= 0, r16 < BKT)
                mi = jnp.where(m, 1, 0)
                incl = plsc.cumsum(mi)
                pos = p + (incl - mi)
                plsc.store_scatter(lrow_b, [pos], r16, mask=m)
                plsc.store_scatter(gidx_b, [pos], g16, mask=m)
                return p + jnp.max(incl)

            return grp

        cnt = lax.fori_loop(0, NSCH, comp_chunk, 0)

        trash = jnp.full((16,), BKT, jnp.int32)
        gpad = jnp.full((16,), 0, jnp.int32) + goff
        for j in range(CH // 16):
            lrow_b[pl.ds(cnt + j * 16, 16)] = trash
            gidx_b[pl.ds(cnt + j * 16, 16)] = gpad

        pltpu.sync_copy(z128_hbm, accA.at[pl.ds(s * ZPT_A, ZPT_A)])
        pltpu.sync_copy(z128_hbm, accB.at[pl.ds(s * ZPT_A, ZPT_A)])
        plsc.subcore_barrier()

        nch = (cnt + CH - 1) // CH
        bufs = (buf, buf2)
        gidxs = (gidx_v, gidx_v2)
        gsems = (sem1, sem2)

        def fill_idx(kchunk, par):
            o = kchunk * CH

            @plsc.parallel_loop(0, CH // 16, unroll=8)
            def cp(j):
                oj = pl.multiple_of(j * 16, 16)
                gidxs[par][pl.ds(oj, 16)] = gidx_b[pl.ds(o + oj, 16)]

            pltpu.async_copy(tbl_hbm.at[gidxs[par]], bufs[par], gsems[par])

        @pl.when(nch > 0)
        def _():
            fill_idx(0, 0)

        def consume(k, par):
            o = k * CH

            @plsc.parallel_loop(0, CH // 16, unroll=8)
            def cp(j):
                oj = pl.multiple_of(j * 16, 16)
                row_v[pl.ds(oj, 16)] = lrow_b[pl.ds(o + oj, 16)]
            cbuf = bufs[par]
            pltpu.make_async_copy(tbl_hbm.at[gidxs[par]], cbuf,
                                  gsems[par]).wait()

            @plsc.parallel_loop(0, CH, unroll=4)
            def exp_body(e):
                for j in range(4):
                    oj = pl.multiple_of(j * 16, 16)
                    z = cbuf[e, pl.ds(64 + oj, 16)]
                    ex = jnp.exp(z)
                    cbuf[e, pl.ds(64 + oj, 16)] = ex
                    rexbuf[e, pl.ds(oj, 16)] = z * ex
            s1 = pltpu.async_copy(cbuf, accA.at[row_v], sems1, add=True)
            s2 = pltpu.async_copy(rexbuf, accB.at[row_v], sems2, add=True)
            s1.wait()
            s2.wait()

        def pair(kk, _):
            for par in range(2):
                k = 2 * kk + par

                @pl.when(k < nch)
                def _(k=k, par=par):
                    @pl.when(k + 1 < nch)
                    def _():
                        fill_idx(k + 1, 1 - par)

                    consume(k, par)

            return 0

        lax.fori_loop(0, (nch + 1) // 2, pair, 0)
        plsc.subcore_barrier()
        obase = (c * NB + b) * ACC_R + s * ZPT_A
        pltpu.sync_copy(accA.at[pl.ds(s * ZPT_A, ZPT_A)],
                        outA.at[pl.ds(obase, ZPT_A)])
        pltpu.sync_copy(accB.at[pl.ds(s * ZPT_A, ZPT_A)],
                        outB.at[pl.ds(obase, ZPT_A)])
        plsc.subcore_barrier()


def _gather_body(g_hbm, idx_hbm, out_hbm, idx_v, buf, sem):
    c = lax.axis_index("c")
    s = lax.axis_index("s")
    base = (c * NS + s) * (NIDX // (NC * NS))
    pltpu.sync_copy(idx_hbm.at[pl.ds(base, NIDX // (NC * NS))], idx_v)
    pltpu.async_copy(g_hbm.at[idx_v], buf, sem).wait()
    pltpu.sync_copy(buf, out_hbm.at[pl.ds(base, NIDX // (NC * NS))])


def _sc_calls():
    f32 = jnp.float32
    i32 = jnp.int32
    mesh = plsc.VectorSubcoreMesh(core_axis_name="c", subcore_axis_name="s",
                                  num_cores=NC, num_subcores=NS)
    deg = pl.kernel(
        _deg_body,
        out_type=jax.ShapeDtypeStruct((NC * NS * N,), f32),
        mesh=mesh,
        compiler_params=pltpu.CompilerParams(needs_layout_passes=False),
        scratch_types=[
            pltpu.VMEM((SCH,), i32),
            pltpu.VMEM((N,), f32),
        ],
    )
    edge = pl.kernel(
        _edge_body,
        out_type=(jax.ShapeDtypeStruct((NC * NB * ACC_R, D), f32),
                  jax.ShapeDtypeStruct((NC * NB * ACC_R, D), f32)),
        mesh=mesh,
        compiler_params=pltpu.CompilerParams(needs_layout_passes=False),
        scratch_types=[
            pltpu.VMEM((SCH,), i32), pltpu.VMEM((SCH,), i32),
            pltpu.VMEM((SCH,), i32),
            pltpu.VMEM((CAP,), i32), pltpu.VMEM((CAP,), i32),
            pltpu.VMEM((CH,), i32), pltpu.VMEM((CH,), i32),
            pltpu.VMEM((CH,), i32),
            pltpu.VMEM((CH, D), f32), pltpu.VMEM((CH, D), f32),
            pltpu.VMEM((CH, D), f32),
            pltpu.VMEM_SHARED((ACC_R, D), f32),
            pltpu.VMEM_SHARED((ACC_R, D), f32),
            pltpu.SemaphoreType.DMA, pltpu.SemaphoreType.DMA,
            pltpu.SemaphoreType.DMA, pltpu.SemaphoreType.DMA,
        ],
    )
    gather = pl.kernel(
        _gather_body,
        out_type=jax.ShapeDtypeStruct((NIDX, D), f32),
        mesh=mesh,
        scratch_types=[
            pltpu.VMEM((NIDX // (NC * NS),), i32),
            pltpu.VMEM((NIDX // (NC * NS), D), f32),
            pltpu.SemaphoreType.DMA,
        ],
    )
    return deg, edge, gather


def kernel(x, edge_index, idx, edge_type, edge_weight, proj_W, proj_b,
           bn_gamma, bn_beta, W_rel_0, Win_W_0, Win_b_0, Wout_W_0, Wout_b_0,
           W_rel_1, Win_W_1, Win_b_1, Wout_W_1, Wout_b_1):
    f32 = jnp.float32
    i32 = jnp.int32
    proj, prologue_tables, mid, final = _tc_calls()
    deg_k, edge_k, gather_k = _sc_calls()

    row = edge_index[0]
    col = edge_index[1]

    deg_out = deg_k(col)
    degp = deg_out.reshape(NC * NS, N).T

    h, stats = proj(x, proj_W, proj_b.reshape(1, D))
    z128 = jnp.zeros((ZPT_A, D), f32)

    def layer(tbl):
        aA, aB = edge_k(tbl.reshape(NC * REL * N, D), row, col, edge_type,
                        z128)

        def halves(a):
            def core(ci):
                o = ci * NB * ACC_R
                return jnp.concatenate(
                    [a[o + b * ACC_R:o + b * ACC_R + BKT] for b in range(NB)])

            return jnp.stack([core(0), core(1)])

        return halves(aA), halves(aB)

    tbl1 = prologue_tables(h, stats, bn_gamma.reshape(1, D),
                           bn_beta.reshape(1, D), Win_W_0,
                           Win_b_0.reshape(1, D), W_rel_0, degp)
    aA, aB = layer(tbl1)
    tbl2 = mid(aA, aB, degp, Wout_W_0, Wout_b_0.reshape(1, D),
               Win_W_1, Win_b_1.reshape(1, D), W_rel_1)
    aA, aB = layer(tbl2)
    g = final(aA, aB, degp, Wout_W_1, Wout_b_1.reshape(1, D))
    return gather_k(g, idx)
